# baseline (speedup 1.0000x reference)
import functools

import jax
import jax.numpy as jnp
from jax import lax
from jax.experimental import pallas as pl
from jax.experimental.pallas import tpu as pltpu
from jax.experimental.pallas import tpu_sc as plsc

N = 10000
E = 160000
D = 256
H = 256
HH = 128
G = 64
L = 128

NC = 2
NS = 16
CH = 64

NP = 10240
RPT = NP // NS
EP = 163840
EPT = EP // NS
NCH = EPT // CH
EPW = EP // (NC * NS)

BN = 1024
NB = NP // BN

_mesh = plsc.VectorSubcoreMesh(core_axis_name="c", subcore_axis_name="s")


DCH = EPW // CH


def _deg_body(dstp3, ewp3, zcol, out, deg_sh, dst_all, ew_all):
    c = lax.axis_index("c")
    s = lax.axis_index("s")
    w = c * NS + s
    off = s * RPT
    pltpu.sync_copy(zcol.at[pl.ds(off, RPT)], deg_sh.at[pl.ds(off, RPT)])
    pltpu.sync_copy(dstp3.at[pl.ds(w * DCH, DCH)], dst_all)
    pltpu.sync_copy(ewp3.at[pl.ds(w * DCH, DCH)], ew_all)
    plsc.subcore_barrier()

    def chunk(k, carry):
        pltpu.sync_copy(ew_all.at[k], deg_sh.at[dst_all.at[k]], add=True)
        return carry

    lax.fori_loop(0, DCH, chunk, 0)
    plsc.subcore_barrier()
    pltpu.sync_copy(deg_sh.at[pl.ds(off, RPT)], out.at[pl.ds(c * NP + off, RPT)])


_sc_deg = pl.kernel(
    _deg_body,
    out_type=jax.ShapeDtypeStruct((NC * NP,), jnp.float32),
    mesh=_mesh,
    scratch_types=[
        pltpu.VMEM_SHARED((NP,), jnp.float32),
        pltpu.VMEM((DCH, CH), jnp.int32),
        pltpu.VMEM((DCH, CH), jnp.float32),
    ],
)


NBUF = 4


EPC = EP // CH


def _edge_body(combo, ewp, table, zrows, out,
               acc_sh, ib0, ib1, ib2, ib3, wv0, wv1, wv2, wv3,
               rows0, rows1, rows2, rows3,
               is0, is1, is2, is3, gs0, gs1, gs2, gs3, ss0, ss1, ss2, ss3):
    c = lax.axis_index("c")
    s = lax.axis_index("s")
    off = s * RPT
    pltpu.sync_copy(zrows.at[pl.ds(off, RPT)], acc_sh.at[pl.ds(off, RPT)])

    cbase = c * EPC + s * NCH
    plsc.subcore_barrier()

    rows = (rows0, rows1, rows2, rows3)
    ibuf = (ib0, ib1, ib2, ib3)
    ewv = (wv0, wv1, wv2, wv3)
    isem = (is0, is1, is2, is3)
    gsem = (gs0, gs1, gs2, gs3)
    ssem = (ss0, ss1, ss2, ss3)

    ebase = s * EPT

    def idescs(k, b):
        return (
            pltpu.make_async_copy(combo.at[cbase + k], ibuf[b], isem[b]),
            pltpu.make_async_copy(
                ewp.at[pl.ds(ebase + k * CH, CH)], ewv[b], isem[b]),
        )

    def istart(k, b):
        for d in idescs(k, b):
            d.start()

    def iwait(k, b):
        for d in idescs(k, b):
            d.wait()

    def gdesc(k, b):
        return pltpu.make_async_copy(table.at[ibuf[b].at[0]], rows[b], gsem[b])

    def sdesc(k, b):
        return pltpu.make_async_copy(rows[b], acc_sh.at[ibuf[b].at[1]], ssem[b])

    def scale(b):
        rbuf = rows[b]

        def gbody(g, c2):
            wv = ewv[b][pl.ds(g * 16, 16)]
            for lane in range(16):
                wsp = lax.gather(
                    wv, jnp.full((16, 1), lane, jnp.int32),
                    lax.GatherDimensionNumbers(
                        offset_dims=(), collapsed_slice_dims=(0,),
                        start_index_map=(0,)),
                    (1,), mode=lax.GatherScatterMode.PROMISE_IN_BOUNDS)
                e = g * 16 + lane
                for v in range(8):
                    sl = pl.ds(v * 16, 16)
                    rbuf[e, sl] = rbuf[e, sl] * wsp
            return c2

        lax.fori_loop(0, CH // 16, gbody, 0)

    istart(0, 0)
    istart(1, 1)
    iwait(0, 0)
    gdesc(0, 0).start()

    def outer(kk, carry):
        for b in range(NBUF):
            k = kk * NBUF + b

            @pl.when(k > NBUF - 2)
            def _():
                sdesc(k - (NBUF - 1), (b + 1) % NBUF).wait()

            @pl.when(k + 1 < NCH)
            def _():
                iwait(k + 1, (b + 1) % NBUF)
                gdesc(k + 1, (b + 1) % NBUF).start()

            gdesc(k, b).wait()
            scale(b)

            @pl.when(k + 2 < NCH)
            def _():
                istart(k + 2, (b + 2) % NBUF)

            sdesc(k, b).start(add=True)
        return carry

    lax.fori_loop(0, NCH // NBUF, outer, 0)
    for j in range(NBUF - 1):
        sdesc(NCH - (NBUF - 1) + j, (NCH - (NBUF - 1) + j) % NBUF).wait()
    plsc.subcore_barrier()
    pltpu.sync_copy(acc_sh.at[pl.ds(off, RPT)], out.at[pl.ds(c * NP + off, RPT)])


_sc_edge = pl.kernel(
    _edge_body,
    out_type=jax.ShapeDtypeStruct((NC * NP, HH), jnp.float32),
    mesh=_mesh,
    scratch_types=(
        [pltpu.VMEM_SHARED((NP, HH), jnp.float32)]
        + [pltpu.VMEM((2, CH), jnp.int32) for _ in range(NBUF)]
        + [pltpu.VMEM((CH,), jnp.float32) for _ in range(NBUF)]
        + [pltpu.VMEM((CH, HH), jnp.float32) for _ in range(NBUF)]
        + [pltpu.SemaphoreType.DMA for _ in range(3 * NBUF)]
    ),
)


def _dis_block(dlo_ref, dhi_ref):
    degt = 1.0 + dlo_ref[0, 0, :] + dhi_ref[0, 0, :]
    return jnp.where(degt > 0, lax.rsqrt(degt), 0.0)


def _mm0_body(x_ref, w_ref, dlo_ref, dhi_ref, o_ref):
    dis = _dis_block(dlo_ref, dhi_ref)
    xw = jnp.dot(x_ref[...], w_ref[...], preferred_element_type=jnp.float32)
    o_ref[...] = xw * dis[:, None]


def _tc_mm0(xp, W0, deg2):
    return pl.pallas_call(
        _mm0_body,
        grid=(NC, NB),
        in_specs=[
            pl.BlockSpec((BN, D), lambda j, i: (i, 0)),
            pl.BlockSpec((D, HH), lambda j, i: (0, j)),
            pl.BlockSpec((1, 1, BN), lambda j, i: (i, 0, 0)),
            pl.BlockSpec((1, 1, BN), lambda j, i: (NB + i, 0, 0)),
        ],
        out_specs=pl.BlockSpec((BN, HH), lambda j, i: (j * NB + i, 0)),
        out_shape=jax.ShapeDtypeStruct((NC * NP, HH), jnp.float32),
    )(xp, W0, deg2, deg2)


def _mid_body(alo_ref, ahi_ref, tlo_ref, thi_ref, w_ref, b_ref,
              dlo_ref, dhi_ref, o_ref):
    dis = _dis_block(dlo_ref, dhi_ref)
    pre = jnp.concatenate(
        [alo_ref[...] + tlo_ref[...], ahi_ref[...] + thi_ref[...]], axis=1)
    h = jnp.maximum(pre * dis[:, None] + b_ref[...][None, :], 0.0)
    o_ref[...] = jnp.dot(h, w_ref[...],
                         preferred_element_type=jnp.float32) * dis[:, None]


def _tc_mid(acc0, xw0p, W1, b0, deg2):
    blo = pl.BlockSpec((BN, HH), lambda j, i: (i, 0))
    bhi = pl.BlockSpec((BN, HH), lambda j, i: (NB + i, 0))
    return pl.pallas_call(
        _mid_body,
        grid=(NC, NB),
        in_specs=[
            blo, bhi, blo, bhi,
            pl.BlockSpec((H, HH), lambda j, i: (0, j)),
            pl.BlockSpec((H,), lambda j, i: (0,)),
            pl.BlockSpec((1, 1, BN), lambda j, i: (i, 0, 0)),
            pl.BlockSpec((1, 1, BN), lambda j, i: (NB + i, 0, 0)),
        ],
        out_specs=pl.BlockSpec((BN, HH), lambda j, i: (j * NB + i, 0)),
        out_shape=jax.ShapeDtypeStruct((NC * NP, HH), jnp.float32),
    )(acc0, acc0, xw0p, xw0p, W1, b0, deg2, deg2)


def _pool_body(alo_ref, ahi_ref, tlo_ref, thi_ref, b_ref, dlo_ref, dhi_ref,
               batch_ref, pooled_ref, counts_ref):
    i = pl.program_id(0)
    dis = _dis_block(dlo_ref, dhi_ref)
    pre = jnp.concatenate(
        [alo_ref[...] + tlo_ref[...], ahi_ref[...] + thi_ref[...]], axis=1)
    h = jnp.maximum(pre * dis[:, None] + b_ref[...][None, :], 0.0)
    bt = batch_ref[0, 0, :]
    oh = (bt[:, None] == lax.broadcasted_iota(jnp.int32, (BN, G), 1))
    oh = oh.astype(jnp.float32)
    pp = lax.dot_general(oh, h, (((0,), (0,)), ((), ())),
                         preferred_element_type=jnp.float32)
    cc = jnp.sum(oh, axis=0)

    @pl.when(i == 0)
    def _():
        pooled_ref[...] = pp
        counts_ref[...] = cc

    @pl.when(i != 0)
    def _():
        pooled_ref[...] += pp
        counts_ref[...] += cc


def _tc_pool(acc1, xw1p, b1, deg2, batchp):
    blo = pl.BlockSpec((BN, HH), lambda i: (i, 0))
    bhi = pl.BlockSpec((BN, HH), lambda i: (NB + i, 0))
    return pl.pallas_call(
        _pool_body,
        grid=(NB,),
        in_specs=[
            blo, bhi, blo, bhi,
            pl.BlockSpec((H,), lambda i: (0,)),
            pl.BlockSpec((1, 1, BN), lambda i: (i, 0, 0)),
            pl.BlockSpec((1, 1, BN), lambda i: (NB + i, 0, 0)),
            pl.BlockSpec((1, 1, BN), lambda i: (i, 0, 0)),
        ],
        out_specs=[
            pl.BlockSpec((G, H), lambda i: (0, 0)),
            pl.BlockSpec((G,), lambda i: (0,)),
        ],
        out_shape=[
            jax.ShapeDtypeStruct((G, H), jnp.float32),
            jax.ShapeDtypeStruct((G,), jnp.float32),
        ],
    )(acc1, acc1, xw1p, xw1p, b1, deg2, deg2, batchp)


def _head_body(pooled_ref, counts_ref, wl0_ref, bl0_ref, wl1_ref, bl1_ref,
               o_ref):
    p = pooled_ref[...] / jnp.maximum(counts_ref[...], 1.0)[:, None]
    p = jnp.maximum(p, 0.0)
    h = jnp.dot(p, wl0_ref[...], preferred_element_type=jnp.float32)
    h = jnp.maximum(h + bl0_ref[...][None, :], 0.0)
    o = jnp.dot(h, wl1_ref[...], preferred_element_type=jnp.float32)
    o_ref[...] = o + bl1_ref[...][None, :]


def _tc_head(pooled, counts, Wl0, bl0, Wl1, bl1):
    return pl.pallas_call(
        _head_body,
        grid=(1,),
        in_specs=[
            pl.BlockSpec((G, H), lambda i: (0, 0)),
            pl.BlockSpec((G,), lambda i: (0,)),
            pl.BlockSpec((H, L), lambda i: (0, 0)),
            pl.BlockSpec((L,), lambda i: (0,)),
            pl.BlockSpec((L, 1), lambda i: (0, 0)),
            pl.BlockSpec((1,), lambda i: (0,)),
        ],
        out_specs=pl.BlockSpec((G, 1), lambda i: (0, 0)),
        out_shape=jax.ShapeDtypeStruct((G, 1), jnp.float32),
    )(pooled, counts, Wl0, bl0, Wl1, bl1)


@jax.jit
def kernel(x, edge_index, edge_attr, batch, W0, b0, W1, b1, Wl0, bl0, Wl1, bl1):
    src = edge_index[0]
    dst = edge_index[1]
    pad_e = EP - E
    pidx = jnp.arange(pad_e, dtype=jnp.int32) % N
    srcp = jnp.concatenate([src, pidx])
    dstp = jnp.concatenate([dst, pidx])
    ewp = jnp.concatenate([edge_attr, jnp.zeros((pad_e,), jnp.float32)])
    xp = jnp.concatenate([x, jnp.zeros((NP - N, D), jnp.float32)], axis=0)
    batchp = jnp.concatenate([batch, jnp.full((NP - N,), G, jnp.int32)])
    zcol = jnp.zeros((NP,), jnp.float32)
    zrows = jnp.zeros((NP, HH), jnp.float32)

    dst2d = dstp.reshape(EP // CH, CH)
    ew2d = ewp.reshape(EP // CH, CH)
    dstc = dstp.reshape(1, EP // CH, CH)
    combo = jnp.stack(
        [jnp.stack([srcp, srcp + NP]).reshape(NC, EP // CH, CH),
         jnp.broadcast_to(dstc, (NC, EP // CH, CH))],
        axis=2).reshape(NC * (EP // CH), 2, CH)
    deg2 = _sc_deg(dst2d, ew2d, zcol).reshape(NC * NB, 1, BN)
    batchp = batchp.reshape(NB, 1, BN)
    xw0p = _tc_mm0(xp, W0, deg2)
    acc0 = _sc_edge(combo, ewp, xw0p, zrows)
    xw1p = _tc_mid(acc0, xw0p, W1, b0, deg2)
    acc1 = _sc_edge(combo, ewp, xw1p, zrows)
    pooled, counts = _tc_pool(acc1, xw1p, b1, deg2, batchp)
    return _tc_head(pooled, counts, Wl0, bl0, Wl1, bl1)


# speedup vs baseline: 1.2282x; 1.2282x over previous
import functools

import jax
import jax.numpy as jnp
from jax import lax
from jax.experimental import pallas as pl
from jax.experimental.pallas import tpu as pltpu
from jax.experimental.pallas import tpu_sc as plsc

N = 10000
E = 160000
D = 256
H = 256
HH = 128
G = 64
L = 128

NC = 2
NS = 16
CH = 128

NP = 10240
RPT = NP // NS
EP = 163840
EPT = EP // NS
NCH = EPT // CH
EPW = EP // (NC * NS)

BN = 1024
NB = NP // BN

_mesh = plsc.VectorSubcoreMesh(core_axis_name="c", subcore_axis_name="s")


DCH = EPW // CH


def _deg_body(dstp3, ewp3, zcol, out, deg_sh, dst_all, ew_all):
    c = lax.axis_index("c")
    s = lax.axis_index("s")
    w = c * NS + s
    off = s * RPT
    pltpu.sync_copy(zcol.at[pl.ds(off, RPT)], deg_sh.at[pl.ds(off, RPT)])
    pltpu.sync_copy(dstp3.at[pl.ds(w * DCH, DCH)], dst_all)
    pltpu.sync_copy(ewp3.at[pl.ds(w * DCH, DCH)], ew_all)
    plsc.subcore_barrier()

    def chunk(k, carry):
        pltpu.sync_copy(ew_all.at[k], deg_sh.at[dst_all.at[k]], add=True)
        return carry

    lax.fori_loop(0, DCH, chunk, 0)
    plsc.subcore_barrier()
    pltpu.sync_copy(deg_sh.at[pl.ds(off, RPT)], out.at[pl.ds(c * NP + off, RPT)])


_sc_deg = pl.kernel(
    _deg_body,
    out_type=jax.ShapeDtypeStruct((NC * NP,), jnp.float32),
    mesh=_mesh,
    scratch_types=[
        pltpu.VMEM_SHARED((NP,), jnp.float32),
        pltpu.VMEM((DCH, CH), jnp.int32),
        pltpu.VMEM((DCH, CH), jnp.float32),
    ],
)


NBUF = 2


EPC = EP // CH


def _edge_body(src2, dstp, ewp, table, zrows, out,
               acc_sh, sv0, sv1, wv0, wv1, dv0, dv1, rows0, rows1,
               is0, is1, gs0, gs1, ss0, ss1):
    c = lax.axis_index("c")
    s = lax.axis_index("s")
    off = s * RPT
    pltpu.sync_copy(zrows.at[pl.ds(off, RPT)], acc_sh.at[pl.ds(off, RPT)])

    ebase = s * EPT
    plsc.subcore_barrier()

    rows = (rows0, rows1)
    srcv = (sv0, sv1)
    ewv = (wv0, wv1)
    dstv = (dv0, dv1)
    isem = (is0, is1)
    gsem = (gs0, gs1)
    ssem = (ss0, ss1)

    def idescs(k, b):
        return (
            pltpu.make_async_copy(
                src2.at[pl.ds(c * EP + ebase + k * CH, CH)], srcv[b], isem[b]),
            pltpu.make_async_copy(
                ewp.at[pl.ds(ebase + k * CH, CH)], ewv[b], isem[b]),
            pltpu.make_async_copy(
                dstp.at[pl.ds(ebase + k * CH, CH)], dstv[b], isem[b]),
        )

    def istart(k, b):
        for d in idescs(k, b):
            d.start()

    def iwait(k, b):
        for d in idescs(k, b):
            d.wait()

    def gdesc(k, b):
        return pltpu.make_async_copy(table.at[srcv[b]], rows[b], gsem[b])

    def sdesc(k, b):
        return pltpu.make_async_copy(rows[b], acc_sh.at[dstv[b]], ssem[b])

    def scale(b):
        rbuf = rows[b]

        def gbody(g, c2):
            wv = ewv[b][pl.ds(g * 16, 16)]
            for lane in range(16):
                wsp = lax.gather(
                    wv, jnp.full((16, 1), lane, jnp.int32),
                    lax.GatherDimensionNumbers(
                        offset_dims=(), collapsed_slice_dims=(0,),
                        start_index_map=(0,)),
                    (1,), mode=lax.GatherScatterMode.PROMISE_IN_BOUNDS)
                e = g * 16 + lane
                for v in range(8):
                    sl = pl.ds(v * 16, 16)
                    rbuf[e, sl] = rbuf[e, sl] * wsp
            return c2

        lax.fori_loop(0, CH // 16, gbody, 0)

    istart(0, 0)
    istart(1, 1)
    iwait(0, 0)
    gdesc(0, 0).start()

    def outer(kk, carry):
        for b in range(NBUF):
            k = kk * NBUF + b

            @pl.when(k > NBUF - 2)
            def _():
                sdesc(k - (NBUF - 1), (b + 1) % NBUF).wait()

            @pl.when(k + 1 < NCH)
            def _():
                iwait(k + 1, (b + 1) % NBUF)
                gdesc(k + 1, (b + 1) % NBUF).start()

            gdesc(k, b).wait()
            scale(b)

            @pl.when(k + 2 < NCH)
            def _():
                istart(k + 2, (b + 2) % NBUF)

            sdesc(k, b).start(add=True)
        return carry

    lax.fori_loop(0, NCH // NBUF, outer, 0)
    for j in range(NBUF - 1):
        sdesc(NCH - (NBUF - 1) + j, (NCH - (NBUF - 1) + j) % NBUF).wait()
    plsc.subcore_barrier()
    pltpu.sync_copy(acc_sh.at[pl.ds(off, RPT)], out.at[pl.ds(c * NP + off, RPT)])


_sc_edge = pl.kernel(
    _edge_body,
    out_type=jax.ShapeDtypeStruct((NC * NP, HH), jnp.float32),
    mesh=_mesh,
    scratch_types=(
        [pltpu.VMEM_SHARED((NP, HH), jnp.float32)]
        + [pltpu.VMEM((CH,), jnp.int32) for _ in range(NBUF)]
        + [pltpu.VMEM((CH,), jnp.float32) for _ in range(NBUF)]
        + [pltpu.VMEM((CH,), jnp.int32) for _ in range(NBUF)]
        + [pltpu.VMEM((CH, HH), jnp.float32) for _ in range(NBUF)]
        + [pltpu.SemaphoreType.DMA for _ in range(3 * NBUF)]
    ),
)


def _dis_block(dlo_ref, dhi_ref):
    degt = 1.0 + dlo_ref[0, 0, :] + dhi_ref[0, 0, :]
    return jnp.where(degt > 0, lax.rsqrt(degt), 0.0)


def _mm0_body(x_ref, w_ref, dlo_ref, dhi_ref, o_ref):
    dis = _dis_block(dlo_ref, dhi_ref)
    xw = jnp.dot(x_ref[...], w_ref[...], preferred_element_type=jnp.float32)
    o_ref[...] = xw * dis[:, None]


def _tc_mm0(xp, W0, deg2):
    return pl.pallas_call(
        _mm0_body,
        grid=(NC, NB),
        in_specs=[
            pl.BlockSpec((BN, D), lambda j, i: (i, 0)),
            pl.BlockSpec((D, HH), lambda j, i: (0, j)),
            pl.BlockSpec((1, 1, BN), lambda j, i: (i, 0, 0)),
            pl.BlockSpec((1, 1, BN), lambda j, i: (NB + i, 0, 0)),
        ],
        out_specs=pl.BlockSpec((BN, HH), lambda j, i: (j * NB + i, 0)),
        out_shape=jax.ShapeDtypeStruct((NC * NP, HH), jnp.float32),
    )(xp, W0, deg2, deg2)


def _mid_body(alo_ref, ahi_ref, tlo_ref, thi_ref, w_ref, b_ref,
              dlo_ref, dhi_ref, o_ref):
    dis = _dis_block(dlo_ref, dhi_ref)
    pre = jnp.concatenate(
        [alo_ref[...] + tlo_ref[...], ahi_ref[...] + thi_ref[...]], axis=1)
    h = jnp.maximum(pre * dis[:, None] + b_ref[...][None, :], 0.0)
    o_ref[...] = jnp.dot(h, w_ref[...],
                         preferred_element_type=jnp.float32) * dis[:, None]


def _tc_mid(acc0, xw0p, W1, b0, deg2):
    blo = pl.BlockSpec((BN, HH), lambda j, i: (i, 0))
    bhi = pl.BlockSpec((BN, HH), lambda j, i: (NB + i, 0))
    return pl.pallas_call(
        _mid_body,
        grid=(NC, NB),
        in_specs=[
            blo, bhi, blo, bhi,
            pl.BlockSpec((H, HH), lambda j, i: (0, j)),
            pl.BlockSpec((H,), lambda j, i: (0,)),
            pl.BlockSpec((1, 1, BN), lambda j, i: (i, 0, 0)),
            pl.BlockSpec((1, 1, BN), lambda j, i: (NB + i, 0, 0)),
        ],
        out_specs=pl.BlockSpec((BN, HH), lambda j, i: (j * NB + i, 0)),
        out_shape=jax.ShapeDtypeStruct((NC * NP, HH), jnp.float32),
    )(acc0, acc0, xw0p, xw0p, W1, b0, deg2, deg2)


def _pool_body(alo_ref, ahi_ref, tlo_ref, thi_ref, b_ref, dlo_ref, dhi_ref,
               batch_ref, pooled_ref, counts_ref):
    i = pl.program_id(0)
    dis = _dis_block(dlo_ref, dhi_ref)
    pre = jnp.concatenate(
        [alo_ref[...] + tlo_ref[...], ahi_ref[...] + thi_ref[...]], axis=1)
    h = jnp.maximum(pre * dis[:, None] + b_ref[...][None, :], 0.0)
    bt = batch_ref[0, 0, :]
    oh = (bt[:, None] == lax.broadcasted_iota(jnp.int32, (BN, G), 1))
    oh = oh.astype(jnp.float32)
    pp = lax.dot_general(oh, h, (((0,), (0,)), ((), ())),
                         preferred_element_type=jnp.float32)
    cc = jnp.sum(oh, axis=0)

    @pl.when(i == 0)
    def _():
        pooled_ref[...] = pp
        counts_ref[...] = cc

    @pl.when(i != 0)
    def _():
        pooled_ref[...] += pp
        counts_ref[...] += cc


def _tc_pool(acc1, xw1p, b1, deg2, batchp):
    blo = pl.BlockSpec((BN, HH), lambda i: (i, 0))
    bhi = pl.BlockSpec((BN, HH), lambda i: (NB + i, 0))
    return pl.pallas_call(
        _pool_body,
        grid=(NB,),
        in_specs=[
            blo, bhi, blo, bhi,
            pl.BlockSpec((H,), lambda i: (0,)),
            pl.BlockSpec((1, 1, BN), lambda i: (i, 0, 0)),
            pl.BlockSpec((1, 1, BN), lambda i: (NB + i, 0, 0)),
            pl.BlockSpec((1, 1, BN), lambda i: (i, 0, 0)),
        ],
        out_specs=[
            pl.BlockSpec((G, H), lambda i: (0, 0)),
            pl.BlockSpec((G,), lambda i: (0,)),
        ],
        out_shape=[
            jax.ShapeDtypeStruct((G, H), jnp.float32),
            jax.ShapeDtypeStruct((G,), jnp.float32),
        ],
    )(acc1, acc1, xw1p, xw1p, b1, deg2, deg2, batchp)


def _head_body(pooled_ref, counts_ref, wl0_ref, bl0_ref, wl1_ref, bl1_ref,
               o_ref):
    p = pooled_ref[...] / jnp.maximum(counts_ref[...], 1.0)[:, None]
    p = jnp.maximum(p, 0.0)
    h = jnp.dot(p, wl0_ref[...], preferred_element_type=jnp.float32)
    h = jnp.maximum(h + bl0_ref[...][None, :], 0.0)
    o = jnp.dot(h, wl1_ref[...], preferred_element_type=jnp.float32)
    o_ref[...] = o + bl1_ref[...][None, :]


def _tc_head(pooled, counts, Wl0, bl0, Wl1, bl1):
    return pl.pallas_call(
        _head_body,
        grid=(1,),
        in_specs=[
            pl.BlockSpec((G, H), lambda i: (0, 0)),
            pl.BlockSpec((G,), lambda i: (0,)),
            pl.BlockSpec((H, L), lambda i: (0, 0)),
            pl.BlockSpec((L,), lambda i: (0,)),
            pl.BlockSpec((L, 1), lambda i: (0, 0)),
            pl.BlockSpec((1,), lambda i: (0,)),
        ],
        out_specs=pl.BlockSpec((G, 1), lambda i: (0, 0)),
        out_shape=jax.ShapeDtypeStruct((G, 1), jnp.float32),
    )(pooled, counts, Wl0, bl0, Wl1, bl1)


@jax.jit
def kernel(x, edge_index, edge_attr, batch, W0, b0, W1, b1, Wl0, bl0, Wl1, bl1):
    src = edge_index[0]
    dst = edge_index[1]
    pad_e = EP - E
    pidx = jnp.arange(pad_e, dtype=jnp.int32) % N
    srcp = jnp.concatenate([src, pidx])
    dstp = jnp.concatenate([dst, pidx])
    ewp = jnp.concatenate([edge_attr, jnp.zeros((pad_e,), jnp.float32)])
    xp = jnp.concatenate([x, jnp.zeros((NP - N, D), jnp.float32)], axis=0)
    batchp = jnp.concatenate([batch, jnp.full((NP - N,), G, jnp.int32)])
    zcol = jnp.zeros((NP,), jnp.float32)
    zrows = jnp.zeros((NP, HH), jnp.float32)

    dst2d = dstp.reshape(EP // CH, CH)
    ew2d = ewp.reshape(EP // CH, CH)
    src2 = jnp.concatenate([srcp, srcp + NP])
    deg2 = _sc_deg(dst2d, ew2d, zcol).reshape(NC * NB, 1, BN)
    batchp = batchp.reshape(NB, 1, BN)
    xw0p = _tc_mm0(xp, W0, deg2)
    acc0 = _sc_edge(src2, dstp, ewp, xw0p, zrows)
    xw1p = _tc_mid(acc0, xw0p, W1, b0, deg2)
    acc1 = _sc_edge(src2, dstp, ewp, xw1p, zrows)
    pooled, counts = _tc_pool(acc1, xw1p, b1, deg2, batchp)
    return _tc_head(pooled, counts, Wl0, bl0, Wl1, bl1)


# speedup vs baseline: 1.2343x; 1.0050x over previous
import functools

import jax
import jax.numpy as jnp
from jax import lax
from jax.experimental import pallas as pl
from jax.experimental.pallas import tpu as pltpu
from jax.experimental.pallas import tpu_sc as plsc

N = 10000
E = 160000
D = 256
H = 256
HH = 128
G = 64
L = 128

NC = 2
NS = 16
CH = 128

NP = 10240
RPT = NP // NS
EP = 163840
EPT = EP // NS
NCH = EPT // CH
EPW = EP // (NC * NS)

BN = 1024
NB = NP // BN

_mesh = plsc.VectorSubcoreMesh(core_axis_name="c", subcore_axis_name="s")


DCH = EPW // CH


def _deg_body(dstp3, ewp3, zcol, out, deg_sh, dst_all, ew_all):
    c = lax.axis_index("c")
    s = lax.axis_index("s")
    w = c * NS + s
    off = s * RPT
    pltpu.sync_copy(zcol.at[pl.ds(off, RPT)], deg_sh.at[pl.ds(off, RPT)])
    pltpu.sync_copy(dstp3.at[pl.ds(w * DCH, DCH)], dst_all)
    pltpu.sync_copy(ewp3.at[pl.ds(w * DCH, DCH)], ew_all)
    plsc.subcore_barrier()

    def chunk(k, carry):
        pltpu.sync_copy(ew_all.at[k], deg_sh.at[dst_all.at[k]], add=True)
        return carry

    lax.fori_loop(0, DCH, chunk, 0)
    plsc.subcore_barrier()
    pltpu.sync_copy(deg_sh.at[pl.ds(off, RPT)], out.at[pl.ds(c * NP + off, RPT)])


_sc_deg = pl.kernel(
    _deg_body,
    out_type=jax.ShapeDtypeStruct((NC * NP,), jnp.float32),
    mesh=_mesh,
    scratch_types=[
        pltpu.VMEM_SHARED((NP,), jnp.float32),
        pltpu.VMEM((DCH, CH), jnp.int32),
        pltpu.VMEM((DCH, CH), jnp.float32),
    ],
)


NBUF = 2


EPC = EP // CH


def _edge_body(src2, dstp, ewp, table, zrows, out,
               acc_sh, sv0, sv1, wv0, wv1, dv0, dv1, rows0, rows1,
               is0, is1, gs0, gs1, ss0, ss1):
    c = lax.axis_index("c")
    s = lax.axis_index("s")
    off = s * RPT
    pltpu.sync_copy(zrows.at[pl.ds(off, RPT)], acc_sh.at[pl.ds(off, RPT)])

    ebase = s * EPT
    plsc.subcore_barrier()

    rows = (rows0, rows1)
    srcv = (sv0, sv1)
    ewv = (wv0, wv1)
    dstv = (dv0, dv1)
    isem = (is0, is1)
    gsem = (gs0, gs1)
    ssem = (ss0, ss1)

    def idescs(k, b):
        return (
            pltpu.make_async_copy(
                src2.at[pl.ds(c * EP + ebase + k * CH, CH)], srcv[b], isem[b]),
            pltpu.make_async_copy(
                ewp.at[pl.ds(ebase + k * CH, CH)], ewv[b], isem[b]),
            pltpu.make_async_copy(
                dstp.at[pl.ds(ebase + k * CH, CH)], dstv[b], isem[b]),
        )

    def istart(k, b):
        for d in idescs(k, b):
            d.start()

    def iwait(k, b):
        for d in idescs(k, b):
            d.wait()

    def gdesc(k, b):
        return pltpu.make_async_copy(table.at[srcv[b]], rows[b], gsem[b])

    def sdesc(k, b):
        return pltpu.make_async_copy(rows[b], acc_sh.at[dstv[b]], ssem[b])

    def scale(b):
        rbuf = rows[b]

        def gbody(g, c2):
            wv = ewv[b][pl.ds(g * 16, 16)]
            for lane in range(16):
                wsp = lax.gather(
                    wv, jnp.full((16, 1), lane, jnp.int32),
                    lax.GatherDimensionNumbers(
                        offset_dims=(), collapsed_slice_dims=(0,),
                        start_index_map=(0,)),
                    (1,), mode=lax.GatherScatterMode.PROMISE_IN_BOUNDS)
                e = g * 16 + lane
                for v in range(8):
                    sl = pl.ds(v * 16, 16)
                    rbuf[e, sl] = rbuf[e, sl] * wsp
            return c2

        lax.fori_loop(0, CH // 16, gbody, 0)

    istart(0, 0)
    istart(1, 1)
    iwait(0, 0)
    gdesc(0, 0).start()

    def outer(kk, carry):
        for b in range(NBUF):
            k = kk * NBUF + b

            @pl.when(k > NBUF - 2)
            def _():
                sdesc(k - (NBUF - 1), (b + 1) % NBUF).wait()

            @pl.when(k + 1 < NCH)
            def _():
                iwait(k + 1, (b + 1) % NBUF)
                gdesc(k + 1, (b + 1) % NBUF).start()

            gdesc(k, b).wait()
            scale(b)

            @pl.when(k + 2 < NCH)
            def _():
                istart(k + 2, (b + 2) % NBUF)

            sdesc(k, b).start(add=True)
        return carry

    lax.fori_loop(0, NCH // NBUF, outer, 0)
    for j in range(NBUF - 1):
        sdesc(NCH - (NBUF - 1) + j, (NCH - (NBUF - 1) + j) % NBUF).wait()
    plsc.subcore_barrier()
    pltpu.sync_copy(acc_sh.at[pl.ds(off, RPT)], out.at[pl.ds(c * NP + off, RPT)])


_sc_edge = pl.kernel(
    _edge_body,
    out_type=jax.ShapeDtypeStruct((NC * NP, HH), jnp.float32),
    mesh=_mesh,
    scratch_types=(
        [pltpu.VMEM_SHARED((NP, HH), jnp.float32)]
        + [pltpu.VMEM((CH,), jnp.int32) for _ in range(NBUF)]
        + [pltpu.VMEM((CH,), jnp.float32) for _ in range(NBUF)]
        + [pltpu.VMEM((CH,), jnp.int32) for _ in range(NBUF)]
        + [pltpu.VMEM((CH, HH), jnp.float32) for _ in range(NBUF)]
        + [pltpu.SemaphoreType.DMA for _ in range(3 * NBUF)]
    ),
)


def _dis_block(dlo_ref, dhi_ref):
    degt = 1.0 + dlo_ref[0, 0, :] + dhi_ref[0, 0, :]
    return jnp.where(degt > 0, lax.rsqrt(degt), 0.0)


def _mm0_body(x_ref, w_ref, dlo_ref, dhi_ref, o_ref):
    dis = _dis_block(dlo_ref, dhi_ref)
    xw = jnp.dot(x_ref[...].astype(jnp.bfloat16),
                 w_ref[...].astype(jnp.bfloat16),
                 preferred_element_type=jnp.float32)
    o_ref[...] = xw * dis[:, None]


def _tc_mm0(xp, W0, deg2):
    return pl.pallas_call(
        _mm0_body,
        grid=(NC, NB),
        in_specs=[
            pl.BlockSpec((BN, D), lambda j, i: (i, 0)),
            pl.BlockSpec((D, HH), lambda j, i: (0, j)),
            pl.BlockSpec((1, 1, BN), lambda j, i: (i, 0, 0)),
            pl.BlockSpec((1, 1, BN), lambda j, i: (NB + i, 0, 0)),
        ],
        out_specs=pl.BlockSpec((BN, HH), lambda j, i: (j * NB + i, 0)),
        out_shape=jax.ShapeDtypeStruct((NC * NP, HH), jnp.float32),
    )(xp, W0, deg2, deg2)


def _mid_body(alo_ref, ahi_ref, tlo_ref, thi_ref, w_ref, b_ref,
              dlo_ref, dhi_ref, o_ref):
    dis = _dis_block(dlo_ref, dhi_ref)
    pre = jnp.concatenate(
        [alo_ref[...] + tlo_ref[...], ahi_ref[...] + thi_ref[...]], axis=1)
    h = jnp.maximum(pre * dis[:, None] + b_ref[...][None, :], 0.0)
    o_ref[...] = jnp.dot(h.astype(jnp.bfloat16),
                         w_ref[...].astype(jnp.bfloat16),
                         preferred_element_type=jnp.float32) * dis[:, None]


def _tc_mid(acc0, xw0p, W1, b0, deg2):
    blo = pl.BlockSpec((BN, HH), lambda j, i: (i, 0))
    bhi = pl.BlockSpec((BN, HH), lambda j, i: (NB + i, 0))
    return pl.pallas_call(
        _mid_body,
        grid=(NC, NB),
        in_specs=[
            blo, bhi, blo, bhi,
            pl.BlockSpec((H, HH), lambda j, i: (0, j)),
            pl.BlockSpec((H,), lambda j, i: (0,)),
            pl.BlockSpec((1, 1, BN), lambda j, i: (i, 0, 0)),
            pl.BlockSpec((1, 1, BN), lambda j, i: (NB + i, 0, 0)),
        ],
        out_specs=pl.BlockSpec((BN, HH), lambda j, i: (j * NB + i, 0)),
        out_shape=jax.ShapeDtypeStruct((NC * NP, HH), jnp.float32),
    )(acc0, acc0, xw0p, xw0p, W1, b0, deg2, deg2)


def _pool_body(alo_ref, ahi_ref, tlo_ref, thi_ref, b_ref, dlo_ref, dhi_ref,
               batch_ref, wl0_ref, bl0_ref, wl1_ref, bl1_ref,
               o_ref, pooled_ref, counts_ref):
    i = pl.program_id(0)
    dis = _dis_block(dlo_ref, dhi_ref)
    pre = jnp.concatenate(
        [alo_ref[...] + tlo_ref[...], ahi_ref[...] + thi_ref[...]], axis=1)
    h = jnp.maximum(pre * dis[:, None] + b_ref[...][None, :], 0.0)
    bt = batch_ref[0, 0, :]
    oh = (bt[:, None] == lax.broadcasted_iota(jnp.int32, (BN, G), 1))
    oh = oh.astype(jnp.float32)
    pp = lax.dot_general(oh, h, (((0,), (0,)), ((), ())),
                         preferred_element_type=jnp.float32)
    cc = jnp.sum(oh, axis=0)

    @pl.when(i == 0)
    def _():
        pooled_ref[...] = pp
        counts_ref[...] = cc

    @pl.when(i != 0)
    def _():
        pooled_ref[...] += pp
        counts_ref[...] += cc

    @pl.when(i == NB - 1)
    def _():
        p = pooled_ref[...] / jnp.maximum(counts_ref[...], 1.0)[:, None]
        p = jnp.maximum(p, 0.0)
        hh = jnp.dot(p, wl0_ref[...], preferred_element_type=jnp.float32)
        hh = jnp.maximum(hh + bl0_ref[...][None, :], 0.0)
        o = jnp.dot(hh, wl1_ref[...], preferred_element_type=jnp.float32)
        o_ref[...] = o + bl1_ref[...][None, :]


def _tc_pool(acc1, xw1p, b1, deg2, batchp, Wl0, bl0, Wl1, bl1):
    blo = pl.BlockSpec((BN, HH), lambda i: (i, 0))
    bhi = pl.BlockSpec((BN, HH), lambda i: (NB + i, 0))
    return pl.pallas_call(
        _pool_body,
        grid=(NB,),
        in_specs=[
            blo, bhi, blo, bhi,
            pl.BlockSpec((H,), lambda i: (0,)),
            pl.BlockSpec((1, 1, BN), lambda i: (i, 0, 0)),
            pl.BlockSpec((1, 1, BN), lambda i: (NB + i, 0, 0)),
            pl.BlockSpec((1, 1, BN), lambda i: (i, 0, 0)),
            pl.BlockSpec((H, L), lambda i: (0, 0)),
            pl.BlockSpec((L,), lambda i: (0,)),
            pl.BlockSpec((L, 1), lambda i: (0, 0)),
            pl.BlockSpec((1,), lambda i: (0,)),
        ],
        out_specs=[
            pl.BlockSpec((G, 1), lambda i: (0, 0)),
            pl.BlockSpec((G, H), lambda i: (0, 0)),
            pl.BlockSpec((G,), lambda i: (0,)),
        ],
        out_shape=[
            jax.ShapeDtypeStruct((G, 1), jnp.float32),
            jax.ShapeDtypeStruct((G, H), jnp.float32),
            jax.ShapeDtypeStruct((G,), jnp.float32),
        ],
    )(acc1, acc1, xw1p, xw1p, b1, deg2, deg2, batchp, Wl0, bl0, Wl1, bl1)


@jax.jit
def kernel(x, edge_index, edge_attr, batch, W0, b0, W1, b1, Wl0, bl0, Wl1, bl1):
    src = edge_index[0]
    dst = edge_index[1]
    pad_e = EP - E
    pidx = jnp.arange(pad_e, dtype=jnp.int32) % N
    srcp = jnp.concatenate([src, pidx])
    dstp = jnp.concatenate([dst, pidx])
    ewp = jnp.concatenate([edge_attr, jnp.zeros((pad_e,), jnp.float32)])
    xp = jnp.concatenate([x, jnp.zeros((NP - N, D), jnp.float32)], axis=0)
    batchp = jnp.concatenate([batch, jnp.full((NP - N,), G, jnp.int32)])
    zcol = jnp.zeros((NP,), jnp.float32)
    zrows = jnp.zeros((NP, HH), jnp.float32)

    dst2d = dstp.reshape(EP // CH, CH)
    ew2d = ewp.reshape(EP // CH, CH)
    src2 = jnp.concatenate([srcp, srcp + NP])
    deg2 = _sc_deg(dst2d, ew2d, zcol).reshape(NC * NB, 1, BN)
    batchp = batchp.reshape(NB, 1, BN)
    xw0p = _tc_mm0(xp, W0, deg2)
    acc0 = _sc_edge(src2, dstp, ewp, xw0p, zrows)
    xw1p = _tc_mid(acc0, xw0p, W1, b0, deg2)
    acc1 = _sc_edge(src2, dstp, ewp, xw1p, zrows)
    out, _, _ = _tc_pool(acc1, xw1p, b1, deg2, batchp, Wl0, bl0, Wl1, bl1)
    return out
